# baseline (speedup 1.0000x reference)
import jax
import jax.numpy as jnp
from jax.experimental import pallas as pl
from jax.experimental.pallas import tpu as pltpu

_K = 16
_N = 4096
_C = 128
_R = 1024


def _knn_tile_kernel(x_ref, out_ref, xnf_ref, xnb_ref, nxsq_ref):
    t = pl.program_id(1)

    @pl.when(t == 0)
    def _prologue():
        xb = x_ref[0]
        norm = jnp.sqrt(jnp.sum(xb * xb, axis=1, keepdims=True))
        xn = xb / jnp.maximum(norm, 1e-12)
        xnf_ref[...] = xn
        xnb_ref[...] = xn.astype(jnp.bfloat16)
        nxsq_ref[...] = -jnp.sum(xn * xn, axis=0, keepdims=True)

    xq2 = jnp.transpose(2.0 * xnf_ref[:, pl.ds(t * _R, _R)])

    inner2 = jax.lax.dot_general(
        xq2.astype(jnp.bfloat16), xnb_ref[...],
        dimension_numbers=(((1,), (0,)), ((), ())),
        preferred_element_type=jnp.float32,
    )

    nxsq = nxsq_ref[...]
    nxsq_q = jnp.transpose(nxsq_ref[:, pl.ds(t * _R, _R)])

    s = (nxsq_q + inner2) + nxsq

    lane = jax.lax.broadcasted_iota(jnp.int32, (_R, _N), 1)
    row = jax.lax.broadcasted_iota(jnp.int32, (_R, 1), 0)
    self_idx = row + t * _R

    out_ref[0, 0, :, 0] = self_idx[:, 0]
    out_ref[1, 0, :, :] = jnp.broadcast_to(self_idx, (_R, _K))
    s = jnp.where(lane == self_idx, -jnp.inf, s)

    for k in range(1, _K):
        idx = jnp.argmax(s, axis=1).astype(jnp.int32)
        out_ref[0, 0, :, k] = idx
        s = jnp.where(lane == idx[:, None], -jnp.inf, s)


def kernel(x):
    b = x.shape[0]
    xb = jnp.squeeze(x, axis=-1)
    grid = (b, _N // _R)
    return pl.pallas_call(
        _knn_tile_kernel,
        grid=grid,
        in_specs=[pl.BlockSpec((1, _C, _N), lambda bb, t: (bb, 0, 0))],
        out_specs=pl.BlockSpec((2, 1, _R, _K), lambda bb, t: (0, bb, t, 0)),
        out_shape=jax.ShapeDtypeStruct((2, b, _N, _K), jnp.int32),
        scratch_shapes=[
            pltpu.VMEM((_C, _N), jnp.float32),
            pltpu.VMEM((_C, _N), jnp.bfloat16),
            pltpu.VMEM((1, _N), jnp.float32),
        ],
        compiler_params=pltpu.CompilerParams(
            dimension_semantics=("arbitrary", "arbitrary"),
        ),
    )(xb)


# speedup vs baseline: 1.1668x; 1.1668x over previous
import jax
import jax.numpy as jnp
from jax.experimental import pallas as pl
from jax.experimental.pallas import tpu as pltpu

_K = 16
_N = 4096
_C = 128
_R = 256


def _knn_tile_kernel(x_ref, out_ref, xnf_ref, xnb_ref, nxsq_ref):
    t = pl.program_id(1)

    @pl.when(t == 0)
    def _prologue():
        xb = x_ref[0]
        norm = jnp.sqrt(jnp.sum(xb * xb, axis=1, keepdims=True))
        xn = xb / jnp.maximum(norm, 1e-12)
        xnf_ref[...] = xn
        xnb_ref[...] = xn.astype(jnp.bfloat16)
        nxsq_ref[...] = -jnp.sum(xn * xn, axis=0, keepdims=True)

    xq2 = jnp.transpose(2.0 * xnf_ref[:, pl.ds(t * _R, _R)])

    inner2 = jax.lax.dot_general(
        xq2.astype(jnp.bfloat16), xnb_ref[...],
        dimension_numbers=(((1,), (0,)), ((), ())),
        preferred_element_type=jnp.float32,
    )

    nxsq = nxsq_ref[...]
    nxsq_q = jnp.transpose(nxsq_ref[:, pl.ds(t * _R, _R)])

    s = (nxsq_q + inner2) + nxsq

    lane = jax.lax.broadcasted_iota(jnp.int32, (_R, _N), 1)
    row = jax.lax.broadcasted_iota(jnp.int32, (_R, 1), 0)
    self_idx = row + t * _R

    out_ref[0, 0, :, 0] = self_idx[:, 0]
    out_ref[1, 0, :, :] = jnp.broadcast_to(self_idx, (_R, _K))
    s = jnp.where(lane == self_idx, -jnp.inf, s)

    for k in range(1, _K):
        idx = jnp.argmax(s, axis=1).astype(jnp.int32)
        out_ref[0, 0, :, k] = idx
        s = jnp.where(lane == idx[:, None], -jnp.inf, s)


def kernel(x):
    b = x.shape[0]
    xb = jnp.squeeze(x, axis=-1)
    grid = (b, _N // _R)
    return pl.pallas_call(
        _knn_tile_kernel,
        grid=grid,
        in_specs=[pl.BlockSpec((1, _C, _N), lambda bb, t: (bb, 0, 0))],
        out_specs=pl.BlockSpec((2, 1, _R, _K), lambda bb, t: (0, bb, t, 0)),
        out_shape=jax.ShapeDtypeStruct((2, b, _N, _K), jnp.int32),
        scratch_shapes=[
            pltpu.VMEM((_C, _N), jnp.float32),
            pltpu.VMEM((_C, _N), jnp.bfloat16),
            pltpu.VMEM((1, _N), jnp.float32),
        ],
        compiler_params=pltpu.CompilerParams(
            dimension_semantics=("arbitrary", "arbitrary"),
        ),
    )(xb)


# speedup vs baseline: 1.2497x; 1.0710x over previous
import jax
import jax.numpy as jnp
from jax.experimental import pallas as pl
from jax.experimental.pallas import tpu as pltpu

_K = 16
_N = 4096
_C = 128
_R = 512


def _knn_tile_kernel(x_ref, out_ref, xnf_ref, xnb_ref, nxsq_ref):
    t = pl.program_id(1)

    @pl.when(t == 0)
    def _prologue():
        xb = x_ref[0]
        norm = jnp.sqrt(jnp.sum(xb * xb, axis=1, keepdims=True))
        xn = xb / jnp.maximum(norm, 1e-12)
        xnf_ref[...] = xn
        xnb_ref[...] = xn.astype(jnp.bfloat16)
        nxsq_ref[...] = -jnp.sum(xn * xn, axis=0, keepdims=True)

    xq2 = jnp.transpose(2.0 * xnf_ref[:, pl.ds(t * _R, _R)])

    inner2 = jax.lax.dot_general(
        xq2.astype(jnp.bfloat16), xnb_ref[...],
        dimension_numbers=(((1,), (0,)), ((), ())),
        preferred_element_type=jnp.float32,
    )

    nxsq = nxsq_ref[...]
    nxsq_q = jnp.transpose(nxsq_ref[:, pl.ds(t * _R, _R)])

    s = (nxsq_q + inner2) + nxsq

    lanef = jax.lax.broadcasted_iota(jnp.int32, (_R, _N), 1).astype(jnp.float32)
    row = jax.lax.broadcasted_iota(jnp.int32, (_R, 1), 0)
    self_idx = row + t * _R

    out_ref[0, 0, :, 0] = self_idx[:, 0]
    out_ref[1, 0, :, :] = jnp.broadcast_to(self_idx, (_R, _K))
    s = jnp.where(lanef == self_idx.astype(jnp.float32), -jnp.inf, s)

    for k in range(1, _K):
        m = jnp.max(s, axis=1, keepdims=True)
        idxf = jnp.min(jnp.where(s == m, lanef, float(_N)), axis=1,
                       keepdims=True)
        out_ref[0, 0, :, k] = idxf[:, 0].astype(jnp.int32)
        s = jnp.where(lanef == idxf, -jnp.inf, s)


def kernel(x):
    b = x.shape[0]
    xb = jnp.squeeze(x, axis=-1)
    grid = (b, _N // _R)
    return pl.pallas_call(
        _knn_tile_kernel,
        grid=grid,
        in_specs=[pl.BlockSpec((1, _C, _N), lambda bb, t: (bb, 0, 0))],
        out_specs=pl.BlockSpec((2, 1, _R, _K), lambda bb, t: (0, bb, t, 0)),
        out_shape=jax.ShapeDtypeStruct((2, b, _N, _K), jnp.int32),
        scratch_shapes=[
            pltpu.VMEM((_C, _N), jnp.float32),
            pltpu.VMEM((_C, _N), jnp.bfloat16),
            pltpu.VMEM((1, _N), jnp.float32),
        ],
        compiler_params=pltpu.CompilerParams(
            dimension_semantics=("arbitrary", "arbitrary"),
        ),
    )(xb)
